# Initial kernel scaffold; baseline (speedup 1.0000x reference)
import jax
import jax.numpy as jnp
from jax.experimental import pallas as pl


def kernel(weapon, rank, level, sub_weapon, special_weapon, weapon_range, weapon_power, weapon_rounds_per, weapon_iine, W_weapon, W_rank, W_sub, W_special):
    raise NotImplementedError("write your pallas kernel here")


# speedup vs baseline: 1.9933x; 1.9933x over previous
import functools

import jax
import jax.numpy as jnp
from jax import lax
from jax.experimental import pallas as pl
from jax.experimental.pallas import tpu as pltpu
from jax.experimental.pallas import tpu_sc as plsc

B = 16384
D = 64
NFEAT = 5
OUT_W = 4 * D + NFEAT

NC = 2
NS = 16
L = 16
NW = NC * NS
BPW = B // NW
CH = 128
NCH = BPW // CH


def _body(weapon, rank, sub_w, spec_w, level, wrange, wpower, wrounds,
          wiine, W_weapon, W_rank, W_sub, W_special, out,
          idx_v, rows_a, rows_b, feats_v, sbuf_v, gsem, osem):
  wid = lax.axis_index("s") * NC + lax.axis_index("c")
  base = wid * BPW

  pltpu.sync_copy(weapon.at[pl.ds(base, BPW)], idx_v.at[0])
  pltpu.sync_copy(rank.at[pl.ds(base, BPW)], idx_v.at[1])
  pltpu.sync_copy(sub_w.at[pl.ds(base, BPW)], idx_v.at[2])
  pltpu.sync_copy(spec_w.at[pl.ds(base, BPW)], idx_v.at[3])

  tables = (W_weapon, W_rank, W_sub, W_special)
  bufs = (rows_a, rows_b)
  out_dma = None
  for t in range(4):
    buf = bufs[t % 2]
    copies = [
        pltpu.async_copy(
            tables[t].at[idx_v.at[t, pl.ds(j * CH, CH)]],
            buf.at[pl.ds(j * CH, CH), :], gsem)
        for j in range(NCH)
    ]
    for c in copies:
      c.wait()
    if out_dma is not None:
      out_dma.wait()
    out_dma = pltpu.async_copy(
        buf, out.at[pl.ds(base, BPW), pl.ds(t * D, D)], osem)

  scalars = (level, wrange, wpower, wrounds, wiine)
  for f in range(NFEAT):
    pltpu.sync_copy(scalars[f].at[pl.ds(base, BPW)], feats_v.at[f])
  for f in range(NFEAT):
    col = jnp.full((L,), f, jnp.int32)
    for j in range(BPW // L):
      vals = feats_v[f, pl.ds(j * L, L)]
      rows = lax.iota(jnp.int32, L) + (j * L)
      plsc.store_scatter(sbuf_v, [rows, col], vals)
  out_dma.wait()
  pltpu.sync_copy(sbuf_v, out.at[pl.ds(base, BPW), pl.ds(4 * D, NFEAT)])


_embed = functools.partial(
    pl.kernel,
    out_type=jax.ShapeDtypeStruct((B, OUT_W), jnp.float32),
    mesh=plsc.VectorSubcoreMesh(core_axis_name="c", subcore_axis_name="s"),
    compiler_params=pltpu.CompilerParams(use_tc_tiling_on_sc=False,
                                        needs_layout_passes=False),
    scratch_types=[
        pltpu.VMEM((4, BPW), jnp.int32),
        pltpu.VMEM((BPW, D), jnp.float32),
        pltpu.VMEM((BPW, D), jnp.float32),
        pltpu.VMEM((NFEAT, BPW), jnp.float32),
        pltpu.VMEM((BPW, NFEAT), jnp.float32),
        pltpu.SemaphoreType.DMA,
        pltpu.SemaphoreType.DMA,
    ],
)(_body)


def kernel(weapon, rank, level, sub_weapon, special_weapon, weapon_range,
           weapon_power, weapon_rounds_per, weapon_iine,
           W_weapon, W_rank, W_sub, W_special):
  return _embed(weapon, rank, sub_weapon, special_weapon, level,
                weapon_range, weapon_power, weapon_rounds_per, weapon_iine,
                W_weapon, W_rank, W_sub, W_special)
